# Initial kernel scaffold; baseline (speedup 1.0000x reference)
import jax
import jax.numpy as jnp
from jax.experimental import pallas as pl


def kernel(x, eW1, eb1, eW2, eb2, eW3, eb3, dW1, db1, dW2, db2, dW3, db3, cb0, cb1, cb2):
    raise NotImplementedError("write your pallas kernel here")


# speedup vs baseline: 2.6951x; 2.6951x over previous
import functools

import jax
import jax.numpy as jnp
from jax.experimental import pallas as pl
from jax.experimental.pallas import tpu as pltpu

_B = 16384
_D = 256
_H = 256
_L = 32
_K = 512
_BETA = 0.25
_BT = 1024


def _fused_body(x_ref, eW1_ref, eb1_ref, eW2_ref, eb2_ref, eW3_ref, eb3_ref,
                dW1_ref, db1_ref, dW2_ref, db2_ref, dW3_ref, db3_ref,
                cb0_ref, cb1_ref, cb2_ref,
                recon_ref, codes_ref, loss_ref):
    f32 = jnp.float32
    x = x_ref[...]
    z = jnp.maximum(
        jnp.dot(x, eW1_ref[...], preferred_element_type=f32) + eb1_ref[...], 0.0)
    z = jnp.maximum(
        jnp.dot(z, eW2_ref[...], preferred_element_type=f32) + eb2_ref[...], 0.0)
    z = jnp.dot(z, eW3_ref[...], preferred_element_type=f32) + eb3_ref[...]

    r = z
    quant = jnp.zeros_like(z)
    loss_acc = jnp.float32(0.0)
    for lvl, cb_ref in enumerate((cb0_ref, cb1_ref, cb2_ref)):
        cb = cb_ref[...]
        r2 = jnp.sum(r * r, axis=1, keepdims=True)
        cross = jax.lax.dot_general(
            r, cb, (((1,), (1,)), ((), ())),
            preferred_element_type=f32)
        c2 = jnp.sum(cb * cb, axis=1)[None, :]
        d2 = r2 - 2.0 * cross + c2
        mind = jnp.min(d2, axis=1, keepdims=True)
        iota = jax.lax.broadcasted_iota(jnp.int32, d2.shape, 1)
        code = jnp.min(jnp.where(d2 == mind, iota, _K), axis=1)
        onehot = (iota == code[:, None]).astype(f32)
        e = jnp.dot(onehot, cb, preferred_element_type=f32)
        quant = quant + e
        r = r - e
        loss_acc = loss_acc + jnp.sum(r * r)
        codes_ref[lvl, :] = code

    h = jnp.maximum(
        jnp.dot(quant, dW1_ref[...], preferred_element_type=f32) + db1_ref[...], 0.0)
    h = jnp.maximum(
        jnp.dot(h, dW2_ref[...], preferred_element_type=f32) + db2_ref[...], 0.0)
    recon_ref[...] = (
        jnp.dot(h, dW3_ref[...], preferred_element_type=f32) + db3_ref[...])

    loss_ref[...] = loss_acc.reshape(1, 1, 1)


@jax.jit
def kernel(x, eW1, eb1, eW2, eb2, eW3, eb3, dW1, db1, dW2, db2, dW3, db3,
           cb0, cb1, cb2):
    grid = _B // _BT
    rep = lambda i: (0, 0)

    recon, codes_t, loss_parts = pl.pallas_call(
        _fused_body,
        grid=(grid,),
        in_specs=[
            pl.BlockSpec((_BT, _D), lambda i: (i, 0)),
            pl.BlockSpec((_D, _H), rep),
            pl.BlockSpec((1, _H), rep),
            pl.BlockSpec((_H, _H), rep),
            pl.BlockSpec((1, _H), rep),
            pl.BlockSpec((_H, _L), rep),
            pl.BlockSpec((1, _L), rep),
            pl.BlockSpec((_L, _H), rep),
            pl.BlockSpec((1, _H), rep),
            pl.BlockSpec((_H, _H), rep),
            pl.BlockSpec((1, _H), rep),
            pl.BlockSpec((_H, _D), rep),
            pl.BlockSpec((1, _D), rep),
            pl.BlockSpec((_K, _L), rep),
            pl.BlockSpec((_K, _L), rep),
            pl.BlockSpec((_K, _L), rep),
        ],
        out_specs=[
            pl.BlockSpec((_BT, _D), lambda i: (i, 0)),
            pl.BlockSpec((3, _BT), lambda i: (0, i)),
            pl.BlockSpec((1, 1, 1), lambda i: (i, 0, 0)),
        ],
        out_shape=[
            jax.ShapeDtypeStruct((_B, _D), jnp.float32),
            jax.ShapeDtypeStruct((3, _B), jnp.int32),
            jax.ShapeDtypeStruct((grid, 1, 1), jnp.float32),
        ],
        compiler_params=pltpu.CompilerParams(
            dimension_semantics=("parallel",)),
    )(x, eW1, eb1.reshape(1, -1), eW2, eb2.reshape(1, -1),
      eW3, eb3.reshape(1, -1), dW1, db1.reshape(1, -1),
      dW2, db2.reshape(1, -1), dW3, db3.reshape(1, -1), cb0, cb1, cb2)

    codes = codes_t.T
    loss = jnp.sum(loss_parts) * ((1.0 + _BETA) / (_B * _L))
    return recon, codes, loss


# speedup vs baseline: 2.9930x; 1.1105x over previous
import functools

import jax
import jax.numpy as jnp
from jax.experimental import pallas as pl
from jax.experimental.pallas import tpu as pltpu

_B = 16384
_D = 256
_H = 256
_L = 32
_K = 512
_BETA = 0.25
_BT = 1024


def _fused_body(x_ref, eW1_ref, eb1_ref, eW2_ref, eb2_ref, eW3_ref, eb3_ref,
                dW1_ref, db1_ref, dW2_ref, db2_ref, dW3_ref, db3_ref,
                cb0_ref, cb1_ref, cb2_ref,
                recon_ref, codes_ref, loss_ref):
    f32 = jnp.float32
    x = x_ref[...]
    z = jnp.maximum(
        jnp.dot(x, eW1_ref[...], preferred_element_type=f32) + eb1_ref[...], 0.0)
    z = jnp.maximum(
        jnp.dot(z, eW2_ref[...], preferred_element_type=f32) + eb2_ref[...], 0.0)
    z = jnp.dot(z, eW3_ref[...], preferred_element_type=f32) + eb3_ref[...]

    r = z
    quant = jnp.zeros_like(z)
    loss_acc = jnp.float32(0.0)
    for lvl, cb_ref in enumerate((cb0_ref, cb1_ref, cb2_ref)):
        cb = cb_ref[...]
        r2 = jnp.sum(r * r, axis=1, keepdims=True)
        cross = jax.lax.dot_general(
            r, cb, (((1,), (1,)), ((), ())),
            preferred_element_type=f32)
        c2 = jnp.sum(cb * cb, axis=1)[None, :]
        d2 = r2 - 2.0 * cross + c2
        mind = jnp.min(d2, axis=1, keepdims=True)
        iota_f = jax.lax.broadcasted_iota(jnp.int32, d2.shape, 1).astype(f32)
        code_f = jnp.min(jnp.where(d2 == mind, iota_f, jnp.float32(_K)), axis=1)
        code = code_f.astype(jnp.int32)
        onehot = (iota_f == code_f[:, None]).astype(f32)
        e = jnp.dot(onehot, cb, preferred_element_type=f32)
        quant = quant + e
        r = r - e
        loss_acc = loss_acc + jnp.sum(r * r)
        codes_ref[lvl, :] = code

    h = jnp.maximum(
        jnp.dot(quant, dW1_ref[...], preferred_element_type=f32) + db1_ref[...], 0.0)
    h = jnp.maximum(
        jnp.dot(h, dW2_ref[...], preferred_element_type=f32) + db2_ref[...], 0.0)
    recon_ref[...] = (
        jnp.dot(h, dW3_ref[...], preferred_element_type=f32) + db3_ref[...])

    loss_ref[...] = loss_acc.reshape(1, 1, 1)


@jax.jit
def kernel(x, eW1, eb1, eW2, eb2, eW3, eb3, dW1, db1, dW2, db2, dW3, db3,
           cb0, cb1, cb2):
    grid = _B // _BT
    rep = lambda i: (0, 0)

    recon, codes_t, loss_parts = pl.pallas_call(
        _fused_body,
        grid=(grid,),
        in_specs=[
            pl.BlockSpec((_BT, _D), lambda i: (i, 0)),
            pl.BlockSpec((_D, _H), rep),
            pl.BlockSpec((1, _H), rep),
            pl.BlockSpec((_H, _H), rep),
            pl.BlockSpec((1, _H), rep),
            pl.BlockSpec((_H, _L), rep),
            pl.BlockSpec((1, _L), rep),
            pl.BlockSpec((_L, _H), rep),
            pl.BlockSpec((1, _H), rep),
            pl.BlockSpec((_H, _H), rep),
            pl.BlockSpec((1, _H), rep),
            pl.BlockSpec((_H, _D), rep),
            pl.BlockSpec((1, _D), rep),
            pl.BlockSpec((_K, _L), rep),
            pl.BlockSpec((_K, _L), rep),
            pl.BlockSpec((_K, _L), rep),
        ],
        out_specs=[
            pl.BlockSpec((_BT, _D), lambda i: (i, 0)),
            pl.BlockSpec((3, _BT), lambda i: (0, i)),
            pl.BlockSpec((1, 1, 1), lambda i: (i, 0, 0)),
        ],
        out_shape=[
            jax.ShapeDtypeStruct((_B, _D), jnp.float32),
            jax.ShapeDtypeStruct((3, _B), jnp.int32),
            jax.ShapeDtypeStruct((grid, 1, 1), jnp.float32),
        ],
        compiler_params=pltpu.CompilerParams(
            dimension_semantics=("parallel",)),
    )(x, eW1, eb1.reshape(1, -1), eW2, eb2.reshape(1, -1),
      eW3, eb3.reshape(1, -1), dW1, db1.reshape(1, -1),
      dW2, db2.reshape(1, -1), dW3, db3.reshape(1, -1), cb0, cb1, cb2)

    codes = codes_t.T
    loss = jnp.sum(loss_parts) * ((1.0 + _BETA) / (_B * _L))
    return recon, codes, loss
